# baseline (speedup 1.0000x reference)
import functools

import jax
import jax.numpy as jnp
from jax import lax
from jax.experimental import pallas as pl
from jax.experimental.pallas import tpu as pltpu
from jax.experimental.pallas import tpu_sc as plsc

_NC = 2
_NS = 16
_CH = 96
_BN = 1000


def _sc_mesh():
    return plsc.VectorSubcoreMesh(
        core_axis_name="c", subcore_axis_name="s",
        num_cores=_NC, num_subcores=_NS)


@functools.lru_cache(maxsize=None)
def _make_sc_deg(npad, e):
    ec = e // _CH
    ecc = ec // _NC
    maxch = ecc // _NS
    rpt = npad // _NS

    @functools.partial(
        pl.kernel,
        out_type=jax.ShapeDtypeStruct((_NC, npad, 128), jnp.float32),
        mesh=_sc_mesh(),
        scratch_types=[
            pltpu.VMEM((maxch, 1, _CH), jnp.int32),
            pltpu.VMEM((_CH, 128), jnp.float32),
            pltpu.VMEM_SHARED((npad, 128), jnp.float32),
            pltpu.SemaphoreType.DMA,
        ],
    )
    def sc_deg(col3_hbm, e0src_hbm, zeros_hbm, out_hbm, cidx_v, src_v, acc_sh, sem):
        c = lax.axis_index("c")
        s = lax.axis_index("s")
        r0 = s * rpt
        pltpu.sync_copy(e0src_hbm, src_v)
        pltpu.sync_copy(zeros_hbm.at[pl.ds(r0, rpt)], acc_sh.at[pl.ds(r0, rpt)])
        gstart = c * ecc + s * maxch
        pltpu.sync_copy(col3_hbm.at[pl.ds(gstart, maxch)], cidx_v)
        plsc.subcore_barrier()

        def body(j, carry):
            pltpu.async_copy(src_v, acc_sh.at[cidx_v.at[j, 0]], sem, add=True)
            return carry
        lax.fori_loop(0, maxch, body, 0)

        def drain(j, carry):
            pltpu.make_async_copy(src_v, acc_sh.at[cidx_v.at[0, 0]], sem).wait()
            return carry
        lax.fori_loop(0, maxch, drain, 0)

        plsc.subcore_barrier()
        pltpu.sync_copy(acc_sh.at[pl.ds(r0, rpt)],
                        out_hbm.at[c, pl.ds(r0, rpt)])

    return sc_deg


@functools.lru_cache(maxsize=None)
def _make_sc_spmm(npad, e, d):
    ec = e // _CH
    ecc = ec // _NC
    maxch = ecc // _NS
    rpt = npad // _NS

    @functools.partial(
        pl.kernel,
        out_type=jax.ShapeDtypeStruct((_NC, npad, d), jnp.float32),
        mesh=_sc_mesh(),
        scratch_types=[
            pltpu.VMEM((maxch * _CH,), jnp.int32),
            pltpu.VMEM((maxch, 1, _CH), jnp.int32),
            pltpu.VMEM((_CH, d), jnp.float32),
            pltpu.VMEM((_CH, d), jnp.float32),
            pltpu.VMEM_SHARED((npad, d), jnp.float32),
            pltpu.SemaphoreType.DMA,
            pltpu.SemaphoreType.DMA,
        ],
    )
    def sc_spmm(row_hbm, col3_hbm, table_hbm, zeros_hbm, out_hbm,
                ridx_v, cidx_v, rows0, rows1, acc_sh, sem0, sem1):
        c = lax.axis_index("c")
        s = lax.axis_index("s")
        r0 = s * rpt
        pltpu.sync_copy(zeros_hbm.at[pl.ds(r0, rpt)], acc_sh.at[pl.ds(r0, rpt)])
        gstart = c * ecc + s * maxch
        goff = pl.multiple_of(gstart * _CH, _CH)
        pltpu.sync_copy(row_hbm.at[pl.ds(goff, maxch * _CH)], ridx_v)
        pltpu.sync_copy(col3_hbm.at[pl.ds(gstart, maxch)], cidx_v)
        plsc.subcore_barrier()

        def gather(j, rows_v, sem):
            off = pl.multiple_of(j * _CH, _CH)
            pltpu.async_copy(
                table_hbm.at[ridx_v.at[pl.ds(off, _CH)]], rows_v, sem)

        def gwait(rows_v, sem):
            pltpu.make_async_copy(zeros_hbm.at[pl.ds(0, _CH)], rows_v, sem).wait()

        gather(0, rows0, sem0)

        def pair(k, carry):
            j0 = 2 * k
            gather(j0 + 1, rows1, sem1)
            gwait(rows0, sem0)
            pltpu.sync_copy(rows0, acc_sh.at[cidx_v.at[j0, 0]], add=True)

            @pl.when(k < maxch // 2 - 1)
            def _():
                gather(j0 + 2, rows0, sem0)
            gwait(rows1, sem1)
            pltpu.sync_copy(rows1, acc_sh.at[cidx_v.at[j0 + 1, 0]], add=True)
            return carry
        lax.fori_loop(0, maxch // 2, pair, 0)

        plsc.subcore_barrier()
        pltpu.sync_copy(acc_sh.at[pl.ds(r0, rpt)],
                        out_hbm.at[c, pl.ds(r0, rpt)])

    return sc_spmm


def _dinv_body(d0_ref, d1_ref, o_ref, *, pad, bpre):
    grow = pl.program_id(0) * bpre + lax.broadcasted_iota(jnp.int32, (bpre, 1), 0)
    deg = d0_ref[:, :1] + d1_ref[:, :1] + 1.0 - (grow < pad).astype(jnp.float32)
    o_ref[...] = lax.rsqrt(deg)


def _tc_dinv(d0, d1, pad):
    npad = d0.shape[0]
    bpre = npad // 16
    return pl.pallas_call(
        functools.partial(_dinv_body, pad=pad, bpre=bpre),
        grid=(npad // bpre,),
        in_specs=[pl.BlockSpec((bpre, 128), lambda i: (i, 0))] * 2,
        out_specs=pl.BlockSpec((bpre, 1), lambda i: (i, 0)),
        out_shape=jax.ShapeDtypeStruct((npad, 1), jnp.float32),
    )(d0, d1)


def _pre_body(f_ref, w_ref, dinv_ref, cs_ref, css_ref, g_ref, be_ref, o_ref,
              *, inv_n, nreal, bpre):
    mean = cs_ref[...] * inv_n
    var = css_ref[...] * inv_n - mean * mean
    alpha = lax.rsqrt(var + 1e-5) * g_ref[...]
    gamma = be_ref[...] - mean * alpha
    fn = f_ref[...] * alpha + gamma
    u = dinv_ref[...] * jnp.dot(fn, w_ref[...],
                                preferred_element_type=jnp.float32)
    grow = pl.program_id(0) * bpre + lax.broadcasted_iota(
        jnp.int32, (bpre, 1), 0)
    o_ref[...] = jnp.where(grow < nreal, u, 0.0)


def _tc_pre(f, w, dinv, cs, css, g, be, nreal):
    npad, din = f.shape
    h = w.shape[1]
    bpre = npad // 16
    return pl.pallas_call(
        functools.partial(_pre_body, inv_n=1.0 / nreal, nreal=nreal, bpre=bpre),
        grid=(npad // bpre,),
        in_specs=[
            pl.BlockSpec((bpre, din), lambda i: (i, 0)),
            pl.BlockSpec((din, h), lambda i: (0, 0)),
            pl.BlockSpec((bpre, 1), lambda i: (i, 0)),
            pl.BlockSpec((1, din), lambda i: (0, 0)),
            pl.BlockSpec((1, din), lambda i: (0, 0)),
            pl.BlockSpec((1, din), lambda i: (0, 0)),
            pl.BlockSpec((1, din), lambda i: (0, 0)),
        ],
        out_specs=pl.BlockSpec((bpre, h), lambda i: (i, 0)),
        out_shape=jax.ShapeDtypeStruct((npad, h), jnp.float32),
    )(f, w, dinv, cs, css, g, be)


def _post_body(p0_ref, p1_ref, u_ref, dinv_ref, b_ref, z_ref, cs_ref, css_ref,
               *, nreal, bpre):
    z = jnp.maximum(
        dinv_ref[...] * (p0_ref[...] + p1_ref[...] + u_ref[...]) + b_ref[...],
        0.0)
    grow = pl.program_id(0) * bpre + lax.broadcasted_iota(
        jnp.int32, (bpre, 1), 0)
    z = jnp.where(grow < nreal, z, 0.0)
    z_ref[...] = z

    @pl.when(pl.program_id(0) == 0)
    def _():
        cs_ref[...] = jnp.zeros_like(cs_ref)
        css_ref[...] = jnp.zeros_like(css_ref)

    cs_ref[...] += jnp.sum(z, axis=0, keepdims=True)
    css_ref[...] += jnp.sum(z * z, axis=0, keepdims=True)


def _tc_post(p0, p1, u, dinv, b, nreal):
    npad, h = u.shape
    bpre = npad // 16
    return pl.pallas_call(
        functools.partial(_post_body, nreal=nreal, bpre=bpre),
        grid=(npad // bpre,),
        in_specs=[
            pl.BlockSpec((bpre, h), lambda i: (i, 0)),
            pl.BlockSpec((bpre, h), lambda i: (i, 0)),
            pl.BlockSpec((bpre, h), lambda i: (i, 0)),
            pl.BlockSpec((bpre, 1), lambda i: (i, 0)),
            pl.BlockSpec((1, h), lambda i: (0, 0)),
        ],
        out_specs=[
            pl.BlockSpec((bpre, h), lambda i: (i, 0)),
            pl.BlockSpec((1, h), lambda i: (0, 0)),
            pl.BlockSpec((1, h), lambda i: (0, 0)),
        ],
        out_shape=[
            jax.ShapeDtypeStruct((npad, h), jnp.float32),
            jax.ShapeDtypeStruct((1, h), jnp.float32),
            jax.ShapeDtypeStruct((1, h), jnp.float32),
        ],
    )(p0, p1, u, dinv, b)


def _mid_body(p0_ref, p1_ref, u_ref, dinv_ref, b_ref, w_ref, g_ref, be_ref,
              o_ref, z_scr, cs_scr, css_scr, *, inv_n, nreal, bpre):
    ph = pl.program_id(0)
    i = pl.program_id(1)
    grow = i * bpre + lax.broadcasted_iota(jnp.int32, (bpre, 1), 0)

    @pl.when(ph == 0)
    def _():
        z = jnp.maximum(
            dinv_ref[...] * (p0_ref[...] + p1_ref[...] + u_ref[...])
            + b_ref[...], 0.0)
        z = jnp.where(grow < nreal, z, 0.0)
        z_scr[pl.ds(i * bpre, bpre), :] = z

        @pl.when(i == 0)
        def _():
            cs_scr[...] = jnp.zeros_like(cs_scr)
            css_scr[...] = jnp.zeros_like(css_scr)

        cs_scr[...] += jnp.sum(z, axis=0, keepdims=True)
        css_scr[...] += jnp.sum(z * z, axis=0, keepdims=True)

    @pl.when(ph == 1)
    def _():
        mean = cs_scr[...] * inv_n
        var = css_scr[...] * inv_n - mean * mean
        alpha = lax.rsqrt(var + 1e-5) * g_ref[...]
        gamma = be_ref[...] - mean * alpha
        fn = z_scr[pl.ds(i * bpre, bpre), :] * alpha + gamma
        un = dinv_ref[...] * jnp.dot(fn, w_ref[...],
                                     preferred_element_type=jnp.float32)
        o_ref[...] = jnp.where(grow < nreal, un, 0.0)


def _tc_mid(p0, p1, u, dinv, b, w, g, be, nreal):
    npad, h = u.shape
    bpre = npad // 16
    return pl.pallas_call(
        functools.partial(_mid_body, inv_n=1.0 / nreal, nreal=nreal, bpre=bpre),
        grid=(2, npad // bpre),
        in_specs=[
            pl.BlockSpec((bpre, h), lambda ph, i: (i, 0)),
            pl.BlockSpec((bpre, h), lambda ph, i: (i, 0)),
            pl.BlockSpec((bpre, h), lambda ph, i: (i, 0)),
            pl.BlockSpec((bpre, 1), lambda ph, i: (i, 0)),
            pl.BlockSpec((1, h), lambda ph, i: (0, 0)),
            pl.BlockSpec((h, h), lambda ph, i: (0, 0)),
            pl.BlockSpec((1, h), lambda ph, i: (0, 0)),
            pl.BlockSpec((1, h), lambda ph, i: (0, 0)),
        ],
        out_specs=pl.BlockSpec((bpre, h), lambda ph, i: (i, 0)),
        out_shape=jax.ShapeDtypeStruct((npad, h), jnp.float32),
        scratch_shapes=[
            pltpu.VMEM((npad, h), jnp.float32),
            pltpu.VMEM((1, h), jnp.float32),
            pltpu.VMEM((1, h), jnp.float32),
        ],
    )(p0, p1, u, dinv, b, w, g, be)


def _pool_body(p0_ref, p1_ref, u_ref, dinv_ref, b_ref, bat_ref, wf_ref, bf_ref,
               o_ref, acc, cnt, *, ngrid, ngraphs):
    i = pl.program_id(0)
    hloc = dinv_ref[...] * (p0_ref[...] + p1_ref[...] + u_ref[...]) + b_ref[...]
    m = (bat_ref[...] == lax.broadcasted_iota(
        jnp.int32, (bat_ref.shape[0], ngraphs), 1)).astype(jnp.float32)
    dn = (((0,), (0,)), ((), ()))
    pm = lax.dot_general(m, hloc, dn, preferred_element_type=jnp.float32)
    pc = lax.dot_general(m, jnp.ones_like(hloc), dn,
                         preferred_element_type=jnp.float32)

    @pl.when(i == 0)
    def _():
        acc[...] = jnp.zeros_like(acc)
        cnt[...] = jnp.zeros_like(cnt)

    acc[...] += pm
    cnt[...] += pc

    @pl.when(i == ngrid - 1)
    def _():
        pooled = acc[...] / jnp.maximum(cnt[...], 1.0)
        o_ref[...] = jnp.dot(
            pooled, wf_ref[...], preferred_element_type=jnp.float32) + bf_ref[...]


def _tc_pool(p0, p1, u, dinv, b, bat, wf, bf, ngraphs):
    n, h = u.shape
    co = wf.shape[1]
    ngrid = n // _BN
    return pl.pallas_call(
        functools.partial(_pool_body, ngrid=ngrid, ngraphs=ngraphs),
        grid=(ngrid,),
        in_specs=[
            pl.BlockSpec((_BN, h), lambda i: (i, 0)),
            pl.BlockSpec((_BN, h), lambda i: (i, 0)),
            pl.BlockSpec((_BN, h), lambda i: (i, 0)),
            pl.BlockSpec((_BN, 1), lambda i: (i, 0)),
            pl.BlockSpec((1, h), lambda i: (0, 0)),
            pl.BlockSpec((_BN, 1), lambda i: (i, 0)),
            pl.BlockSpec((h, co), lambda i: (0, 0)),
            pl.BlockSpec((1, co), lambda i: (0, 0)),
        ],
        out_specs=pl.BlockSpec((ngraphs, co), lambda i: (0, 0)),
        out_shape=jax.ShapeDtypeStruct((ngraphs, co), jnp.float32),
        scratch_shapes=[
            pltpu.VMEM((ngraphs, h), jnp.float32),
            pltpu.VMEM((ngraphs, h), jnp.float32),
        ],
    )(p0, p1, u, dinv, b, bat, wf, bf)


def kernel(x, edge_index, batch, W0, b0, g0, beta0, W1, b1, g1, beta1,
           W2, b2, Wf, bf):
    n, d = x.shape
    e = edge_index.shape[1]
    h = W0.shape[1]
    ngraphs = 64
    npad = _NS * (-(-n // (_NS * 8)) * 8)

    nch = -(-e // _CH)
    ecpad = -(-nch // (2 * _NC * _NS)) * (2 * _NC * _NS)
    epad = ecpad * _CH
    pad = epad - e
    rowpad = n + jnp.arange(pad, dtype=jnp.int32) % (npad - n)
    row = jnp.concatenate([edge_index[0], rowpad])
    colpad = jnp.arange(pad, dtype=jnp.int32)
    col3 = jnp.concatenate([edge_index[1], colpad]).reshape(ecpad, 1, _CH)
    zeros_nd = jnp.zeros((npad, d), jnp.float32)
    e0src = jnp.zeros((_CH, 128), jnp.float32).at[:, 0].set(1.0)

    xp = jnp.concatenate([x, jnp.zeros((npad - n, d), jnp.float32)])

    deg = _make_sc_deg(npad, epad)(col3, e0src, zeros_nd)
    dinv = _tc_dinv(deg[0], deg[1], pad)

    cs = jnp.zeros((1, d), jnp.float32)
    css = jnp.full((1, d), n * (1.0 - 1e-5), jnp.float32)
    ones_r = jnp.ones((1, d), jnp.float32)
    zeros_r = jnp.zeros((1, d), jnp.float32)

    spmm = _make_sc_spmm(npad, epad, h)

    u0 = _tc_pre(xp, W0, dinv, cs, css, ones_r, zeros_r, n)
    p = spmm(row, col3, u0, zeros_nd)
    u1 = _tc_mid(p[0], p[1], u0, dinv, b0.reshape(1, h), W1,
                 g0.reshape(1, h), beta0.reshape(1, h), n)
    p = spmm(row, col3, u1, zeros_nd)
    u2 = _tc_mid(p[0], p[1], u1, dinv, b1.reshape(1, h), W2,
                 g1.reshape(1, h), beta1.reshape(1, h), n)
    p = spmm(row, col3, u2, zeros_nd)

    return _tc_pool(p[0], p[1], u2, dinv, b2.reshape(1, h),
                    batch.reshape(n, 1), Wf, bf.reshape(1, -1), ngraphs)


# speedup vs baseline: 1.0351x; 1.0351x over previous
import functools

import jax
import jax.numpy as jnp
from jax import lax
from jax.experimental import pallas as pl
from jax.experimental.pallas import tpu as pltpu
from jax.experimental.pallas import tpu_sc as plsc

_NC = 2
_NS = 16
_CH = 128
_BN = 1000


def _sc_mesh():
    return plsc.VectorSubcoreMesh(
        core_axis_name="c", subcore_axis_name="s",
        num_cores=_NC, num_subcores=_NS)


@functools.lru_cache(maxsize=None)
def _make_sc_deg(npad, e):
    ec = e // _CH
    ecc = ec // _NC
    maxch = ecc // _NS
    rpt = npad // _NS

    @functools.partial(
        pl.kernel,
        out_type=jax.ShapeDtypeStruct((_NC, npad, 128), jnp.float32),
        mesh=_sc_mesh(),
        scratch_types=[
            pltpu.VMEM((maxch, 1, _CH), jnp.int32),
            pltpu.VMEM((_CH, 128), jnp.float32),
            pltpu.VMEM_SHARED((npad, 128), jnp.float32),
            pltpu.SemaphoreType.DMA,
        ],
    )
    def sc_deg(col3_hbm, e0src_hbm, zeros_hbm, out_hbm, cidx_v, src_v, acc_sh, sem):
        c = lax.axis_index("c")
        s = lax.axis_index("s")
        r0 = s * rpt
        pltpu.sync_copy(e0src_hbm, src_v)
        pltpu.sync_copy(zeros_hbm.at[pl.ds(r0, rpt)], acc_sh.at[pl.ds(r0, rpt)])
        gstart = c * ecc + s * maxch
        pltpu.sync_copy(col3_hbm.at[pl.ds(gstart, maxch)], cidx_v)
        plsc.subcore_barrier()

        def body(j, carry):
            pltpu.async_copy(src_v, acc_sh.at[cidx_v.at[j, 0]], sem, add=True)
            return carry
        lax.fori_loop(0, maxch, body, 0)

        def drain(j, carry):
            pltpu.make_async_copy(src_v, acc_sh.at[cidx_v.at[0, 0]], sem).wait()
            return carry
        lax.fori_loop(0, maxch, drain, 0)

        plsc.subcore_barrier()
        pltpu.sync_copy(acc_sh.at[pl.ds(r0, rpt)],
                        out_hbm.at[c, pl.ds(r0, rpt)])

    return sc_deg


@functools.lru_cache(maxsize=None)
def _make_sc_spmm(npad, e, d):
    ec = e // _CH
    ecc = ec // _NC
    maxch = ecc // _NS
    rpt = npad // _NS

    @functools.partial(
        pl.kernel,
        out_type=jax.ShapeDtypeStruct((_NC, npad, d), jnp.float32),
        mesh=_sc_mesh(),
        scratch_types=[
            pltpu.VMEM((maxch * _CH,), jnp.int32),
            pltpu.VMEM((2, 1, _CH), jnp.int32),
            pltpu.VMEM((_CH, d), jnp.float32),
            pltpu.VMEM((_CH, d), jnp.float32),
            pltpu.VMEM_SHARED((npad, d), jnp.float32),
            pltpu.SemaphoreType.DMA,
            pltpu.SemaphoreType.DMA,
            pltpu.SemaphoreType.DMA,
            pltpu.SemaphoreType.DMA,
        ],
    )
    def sc_spmm(row_hbm, col3_hbm, table_hbm, zeros_hbm, out_hbm,
                ridx_v, cidx_v, rows0, rows1, acc_sh, sem0, sem1, semc0, semc1):
        c = lax.axis_index("c")
        s = lax.axis_index("s")
        r0 = s * rpt
        pltpu.sync_copy(zeros_hbm.at[pl.ds(r0, rpt)], acc_sh.at[pl.ds(r0, rpt)])
        gstart = c * ecc + s * maxch
        goff = pl.multiple_of(gstart * _CH, _CH)
        pltpu.sync_copy(row_hbm.at[pl.ds(goff, maxch * _CH)], ridx_v)
        plsc.subcore_barrier()

        def gather(j, rows_v, sem):
            off = pl.multiple_of(j * _CH, _CH)
            pltpu.async_copy(
                table_hbm.at[ridx_v.at[pl.ds(off, _CH)]], rows_v, sem)

        def gwait(rows_v, sem):
            pltpu.make_async_copy(zeros_hbm.at[pl.ds(0, _CH)], rows_v, sem).wait()

        def cload(j, q, sem):
            pltpu.async_copy(col3_hbm.at[pl.ds(gstart + j, 1)],
                             cidx_v.at[pl.ds(q, 1)], sem)

        def cwait(q, sem):
            pltpu.make_async_copy(col3_hbm.at[pl.ds(gstart, 1)],
                                  cidx_v.at[pl.ds(q, 1)], sem).wait()

        gather(0, rows0, sem0)
        cload(0, 0, semc0)
        cload(1, 1, semc1)

        def pair(k, carry):
            j0 = 2 * k
            gather(j0 + 1, rows1, sem1)
            gwait(rows0, sem0)
            cwait(0, semc0)
            pltpu.sync_copy(rows0, acc_sh.at[cidx_v.at[0, 0]], add=True)

            @pl.when(k < maxch // 2 - 1)
            def _():
                gather(j0 + 2, rows0, sem0)
                cload(j0 + 2, 0, semc0)
            gwait(rows1, sem1)
            cwait(1, semc1)
            pltpu.sync_copy(rows1, acc_sh.at[cidx_v.at[1, 0]], add=True)

            @pl.when(k < maxch // 2 - 1)
            def _():
                cload(j0 + 3, 1, semc1)
            return carry
        lax.fori_loop(0, maxch // 2, pair, 0)

        plsc.subcore_barrier()
        pltpu.sync_copy(acc_sh.at[pl.ds(r0, rpt)],
                        out_hbm.at[c, pl.ds(r0, rpt)])

    return sc_spmm


def _dinv_body(d0_ref, d1_ref, o_ref, *, pad, bpre):
    grow = pl.program_id(0) * bpre + lax.broadcasted_iota(jnp.int32, (bpre, 1), 0)
    deg = d0_ref[:, :1] + d1_ref[:, :1] + 1.0 - (grow < pad).astype(jnp.float32)
    o_ref[...] = lax.rsqrt(deg)


def _tc_dinv(d0, d1, pad):
    npad = d0.shape[0]
    bpre = npad // 16
    return pl.pallas_call(
        functools.partial(_dinv_body, pad=pad, bpre=bpre),
        grid=(npad // bpre,),
        in_specs=[pl.BlockSpec((bpre, 128), lambda i: (i, 0))] * 2,
        out_specs=pl.BlockSpec((bpre, 1), lambda i: (i, 0)),
        out_shape=jax.ShapeDtypeStruct((npad, 1), jnp.float32),
    )(d0, d1)


def _pre_body(f_ref, w_ref, dinv_ref, cs_ref, css_ref, g_ref, be_ref, o_ref,
              *, inv_n, nreal, bpre):
    mean = cs_ref[...] * inv_n
    var = css_ref[...] * inv_n - mean * mean
    alpha = lax.rsqrt(var + 1e-5) * g_ref[...]
    gamma = be_ref[...] - mean * alpha
    fn = f_ref[...] * alpha + gamma
    u = dinv_ref[...] * jnp.dot(fn, w_ref[...],
                                preferred_element_type=jnp.float32)
    grow = pl.program_id(0) * bpre + lax.broadcasted_iota(
        jnp.int32, (bpre, 1), 0)
    o_ref[...] = jnp.where(grow < nreal, u, 0.0)


def _tc_pre(f, w, dinv, cs, css, g, be, nreal):
    npad, din = f.shape
    h = w.shape[1]
    bpre = npad // 16
    return pl.pallas_call(
        functools.partial(_pre_body, inv_n=1.0 / nreal, nreal=nreal, bpre=bpre),
        grid=(npad // bpre,),
        in_specs=[
            pl.BlockSpec((bpre, din), lambda i: (i, 0)),
            pl.BlockSpec((din, h), lambda i: (0, 0)),
            pl.BlockSpec((bpre, 1), lambda i: (i, 0)),
            pl.BlockSpec((1, din), lambda i: (0, 0)),
            pl.BlockSpec((1, din), lambda i: (0, 0)),
            pl.BlockSpec((1, din), lambda i: (0, 0)),
            pl.BlockSpec((1, din), lambda i: (0, 0)),
        ],
        out_specs=pl.BlockSpec((bpre, h), lambda i: (i, 0)),
        out_shape=jax.ShapeDtypeStruct((npad, h), jnp.float32),
    )(f, w, dinv, cs, css, g, be)


def _post_body(p0_ref, p1_ref, u_ref, dinv_ref, b_ref, z_ref, cs_ref, css_ref,
               *, nreal, bpre):
    z = jnp.maximum(
        dinv_ref[...] * (p0_ref[...] + p1_ref[...] + u_ref[...]) + b_ref[...],
        0.0)
    grow = pl.program_id(0) * bpre + lax.broadcasted_iota(
        jnp.int32, (bpre, 1), 0)
    z = jnp.where(grow < nreal, z, 0.0)
    z_ref[...] = z

    @pl.when(pl.program_id(0) == 0)
    def _():
        cs_ref[...] = jnp.zeros_like(cs_ref)
        css_ref[...] = jnp.zeros_like(css_ref)

    cs_ref[...] += jnp.sum(z, axis=0, keepdims=True)
    css_ref[...] += jnp.sum(z * z, axis=0, keepdims=True)


def _tc_post(p0, p1, u, dinv, b, nreal):
    npad, h = u.shape
    bpre = npad // 16
    return pl.pallas_call(
        functools.partial(_post_body, nreal=nreal, bpre=bpre),
        grid=(npad // bpre,),
        in_specs=[
            pl.BlockSpec((bpre, h), lambda i: (i, 0)),
            pl.BlockSpec((bpre, h), lambda i: (i, 0)),
            pl.BlockSpec((bpre, h), lambda i: (i, 0)),
            pl.BlockSpec((bpre, 1), lambda i: (i, 0)),
            pl.BlockSpec((1, h), lambda i: (0, 0)),
        ],
        out_specs=[
            pl.BlockSpec((bpre, h), lambda i: (i, 0)),
            pl.BlockSpec((1, h), lambda i: (0, 0)),
            pl.BlockSpec((1, h), lambda i: (0, 0)),
        ],
        out_shape=[
            jax.ShapeDtypeStruct((npad, h), jnp.float32),
            jax.ShapeDtypeStruct((1, h), jnp.float32),
            jax.ShapeDtypeStruct((1, h), jnp.float32),
        ],
    )(p0, p1, u, dinv, b)


def _mid_body(p0_ref, p1_ref, u_ref, dinv_ref, b_ref, w_ref, g_ref, be_ref,
              o_ref, z_scr, cs_scr, css_scr, *, inv_n, nreal, bpre):
    ph = pl.program_id(0)
    i = pl.program_id(1)
    grow = i * bpre + lax.broadcasted_iota(jnp.int32, (bpre, 1), 0)

    @pl.when(ph == 0)
    def _():
        z = jnp.maximum(
            dinv_ref[...] * (p0_ref[...] + p1_ref[...] + u_ref[...])
            + b_ref[...], 0.0)
        z = jnp.where(grow < nreal, z, 0.0)
        z_scr[pl.ds(i * bpre, bpre), :] = z

        @pl.when(i == 0)
        def _():
            cs_scr[...] = jnp.zeros_like(cs_scr)
            css_scr[...] = jnp.zeros_like(css_scr)

        cs_scr[...] += jnp.sum(z, axis=0, keepdims=True)
        css_scr[...] += jnp.sum(z * z, axis=0, keepdims=True)

    @pl.when(ph == 1)
    def _():
        mean = cs_scr[...] * inv_n
        var = css_scr[...] * inv_n - mean * mean
        alpha = lax.rsqrt(var + 1e-5) * g_ref[...]
        gamma = be_ref[...] - mean * alpha
        fn = z_scr[pl.ds(i * bpre, bpre), :] * alpha + gamma
        un = dinv_ref[...] * jnp.dot(fn, w_ref[...],
                                     preferred_element_type=jnp.float32)
        o_ref[...] = jnp.where(grow < nreal, un, 0.0)


def _tc_mid(p0, p1, u, dinv, b, w, g, be, nreal):
    npad, h = u.shape
    bpre = npad // 16
    return pl.pallas_call(
        functools.partial(_mid_body, inv_n=1.0 / nreal, nreal=nreal, bpre=bpre),
        grid=(2, npad // bpre),
        in_specs=[
            pl.BlockSpec((bpre, h), lambda ph, i: (i, 0)),
            pl.BlockSpec((bpre, h), lambda ph, i: (i, 0)),
            pl.BlockSpec((bpre, h), lambda ph, i: (i, 0)),
            pl.BlockSpec((bpre, 1), lambda ph, i: (i, 0)),
            pl.BlockSpec((1, h), lambda ph, i: (0, 0)),
            pl.BlockSpec((h, h), lambda ph, i: (0, 0)),
            pl.BlockSpec((1, h), lambda ph, i: (0, 0)),
            pl.BlockSpec((1, h), lambda ph, i: (0, 0)),
        ],
        out_specs=pl.BlockSpec((bpre, h), lambda ph, i: (i, 0)),
        out_shape=jax.ShapeDtypeStruct((npad, h), jnp.float32),
        scratch_shapes=[
            pltpu.VMEM((npad, h), jnp.float32),
            pltpu.VMEM((1, h), jnp.float32),
            pltpu.VMEM((1, h), jnp.float32),
        ],
    )(p0, p1, u, dinv, b, w, g, be)


def _pool_body(p0_ref, p1_ref, u_ref, dinv_ref, b_ref, bat_ref, wf_ref, bf_ref,
               o_ref, acc, cnt, *, ngrid, ngraphs):
    i = pl.program_id(0)
    hloc = dinv_ref[...] * (p0_ref[...] + p1_ref[...] + u_ref[...]) + b_ref[...]
    m = (bat_ref[...] == lax.broadcasted_iota(
        jnp.int32, (bat_ref.shape[0], ngraphs), 1)).astype(jnp.float32)
    dn = (((0,), (0,)), ((), ()))
    pm = lax.dot_general(m, hloc, dn, preferred_element_type=jnp.float32)
    pc = lax.dot_general(m, jnp.ones_like(hloc), dn,
                         preferred_element_type=jnp.float32)

    @pl.when(i == 0)
    def _():
        acc[...] = jnp.zeros_like(acc)
        cnt[...] = jnp.zeros_like(cnt)

    acc[...] += pm
    cnt[...] += pc

    @pl.when(i == ngrid - 1)
    def _():
        pooled = acc[...] / jnp.maximum(cnt[...], 1.0)
        o_ref[...] = jnp.dot(
            pooled, wf_ref[...], preferred_element_type=jnp.float32) + bf_ref[...]


def _tc_pool(p0, p1, u, dinv, b, bat, wf, bf, ngraphs):
    n, h = u.shape
    co = wf.shape[1]
    ngrid = n // _BN
    return pl.pallas_call(
        functools.partial(_pool_body, ngrid=ngrid, ngraphs=ngraphs),
        grid=(ngrid,),
        in_specs=[
            pl.BlockSpec((_BN, h), lambda i: (i, 0)),
            pl.BlockSpec((_BN, h), lambda i: (i, 0)),
            pl.BlockSpec((_BN, h), lambda i: (i, 0)),
            pl.BlockSpec((_BN, 1), lambda i: (i, 0)),
            pl.BlockSpec((1, h), lambda i: (0, 0)),
            pl.BlockSpec((_BN, 1), lambda i: (i, 0)),
            pl.BlockSpec((h, co), lambda i: (0, 0)),
            pl.BlockSpec((1, co), lambda i: (0, 0)),
        ],
        out_specs=pl.BlockSpec((ngraphs, co), lambda i: (0, 0)),
        out_shape=jax.ShapeDtypeStruct((ngraphs, co), jnp.float32),
        scratch_shapes=[
            pltpu.VMEM((ngraphs, h), jnp.float32),
            pltpu.VMEM((ngraphs, h), jnp.float32),
        ],
    )(p0, p1, u, dinv, b, bat, wf, bf)


def kernel(x, edge_index, batch, W0, b0, g0, beta0, W1, b1, g1, beta1,
           W2, b2, Wf, bf):
    n, d = x.shape
    e = edge_index.shape[1]
    h = W0.shape[1]
    ngraphs = 64
    npad = _NS * (-(-n // (_NS * 8)) * 8)

    nch = -(-e // _CH)
    ecpad = -(-nch // (2 * _NC * _NS)) * (2 * _NC * _NS)
    epad = ecpad * _CH
    pad = epad - e
    rowpad = n + jnp.arange(pad, dtype=jnp.int32) % (npad - n)
    row = jnp.concatenate([edge_index[0], rowpad])
    colpad = jnp.arange(pad, dtype=jnp.int32)
    col3 = jnp.concatenate([edge_index[1], colpad]).reshape(ecpad, 1, _CH)
    zeros_nd = jnp.zeros((npad, d), jnp.float32)
    e0src = jnp.zeros((_CH, 128), jnp.float32).at[:, 0].set(1.0)

    xp = jnp.concatenate([x, jnp.zeros((npad - n, d), jnp.float32)])

    deg = _make_sc_deg(npad, epad)(col3, e0src, zeros_nd)
    dinv = _tc_dinv(deg[0], deg[1], pad)

    cs = jnp.zeros((1, d), jnp.float32)
    css = jnp.full((1, d), n * (1.0 - 1e-5), jnp.float32)
    ones_r = jnp.ones((1, d), jnp.float32)
    zeros_r = jnp.zeros((1, d), jnp.float32)

    spmm = _make_sc_spmm(npad, epad, h)

    u0 = _tc_pre(xp, W0, dinv, cs, css, ones_r, zeros_r, n)
    p = spmm(row, col3, u0, zeros_nd)
    u1 = _tc_mid(p[0], p[1], u0, dinv, b0.reshape(1, h), W1,
                 g0.reshape(1, h), beta0.reshape(1, h), n)
    p = spmm(row, col3, u1, zeros_nd)
    u2 = _tc_mid(p[0], p[1], u1, dinv, b1.reshape(1, h), W2,
                 g1.reshape(1, h), beta1.reshape(1, h), n)
    p = spmm(row, col3, u2, zeros_nd)

    return _tc_pool(p[0], p[1], u2, dinv, b2.reshape(1, h),
                    batch.reshape(n, 1), Wf, bf.reshape(1, -1), ngraphs)
